# baseline (speedup 1.0000x reference)
import functools

import jax
import jax.numpy as jnp
import numpy as np
from jax import lax
from jax.experimental import pallas as pl
from jax.experimental.pallas import tpu as pltpu
from jax.experimental.pallas import tpu_sc as plsc

HIDDEN = 128
NUM_EMB = 16
_POW1 = np.array([10.0 ** i for i in range(NUM_EMB)], dtype=np.float32)
_RCP1 = np.float32(1.0) / _POW1
_SCL1 = _RCP1 * np.float32(0.0625)

_NW = 32
_CHUNK = 512
_CHAIN_LO, _CHAIN_HI = 1, 9


def _splat(vec, t):
    idx = jnp.broadcast_to(t, (16,)).astype(jnp.int32)
    dn = lax.GatherDimensionNumbers(
        offset_dims=(), collapsed_slice_dims=(0,), start_index_map=(0,))
    return lax.gather(vec, idx[:, None], dn, slice_sizes=(1,),
                      mode=lax.GatherScatterMode.PROMISE_IN_BOUNDS)


def _sc_body(n_tokens, nums_hbm, wtab_hbm, consts_hbm, out_hbm,
             nums_v, wtab_v, consts_v, out_v):
    tokens_pw = n_tokens // _NW
    wid = lax.axis_index("s") * 2 + lax.axis_index("c")
    pltpu.sync_copy(wtab_hbm, wtab_v)
    pltpu.sync_copy(consts_hbm, consts_v)
    pw_vec = consts_v[0, :]
    rc_vec = consts_v[1, :]

    def chunk_body(ch, carry):
        base = wid * tokens_pw + ch * _CHUNK
        pltpu.sync_copy(nums_hbm.at[pl.ds(base, _CHUNK)], nums_v)

        def grp_body(g, c2):
            xi = nums_v[pl.ds(g * 16, 16)]
            xf = xi.astype(jnp.float32)
            for t in range(16):
                x_spl = _splat(xf, jnp.int32(t))
                tq = x_spl * rc_vec
                q = tq.astype(jnp.int32).astype(jnp.float32)
                r = x_spl - q * pw_vec
                r = jnp.where(r == pw_vec, jnp.float32(0.0), r)
                co = jnp.abs(r)
                spl = [_splat(co, jnp.int32(i))
                       for i in range(_CHAIN_LO, _CHAIN_HI + 1)]
                for h in range(HIDDEN // 16):
                    acc = x_spl * wtab_v[10, pl.ds(h * 16, 16)]
                    for k, i in enumerate(range(_CHAIN_LO, _CHAIN_HI + 1)):
                        acc = acc + spl[k] * wtab_v[i, pl.ds(h * 16, 16)]
                    out_v[g * 16 + t, pl.ds(h * 16, 16)] = acc
            return c2

        lax.fori_loop(0, _CHUNK // 16, grp_body, 0, unroll=False)
        pltpu.sync_copy(out_v, out_hbm.at[pl.ds(base, _CHUNK)])
        return carry

    lax.fori_loop(0, tokens_pw // _CHUNK, chunk_body, 0, unroll=False)


@jax.jit
def kernel(numbers, value_embs):
    b, l = numbers.shape
    n = b * l
    nums1 = numbers.reshape(n)
    wtab = value_embs * jnp.asarray(_SCL1)[:, None]
    lin = jnp.sum(wtab[10:16], axis=0)
    wtab = wtab.at[10].set(lin)
    consts = jnp.stack([jnp.asarray(_POW1), jnp.asarray(_RCP1)])
    mesh = plsc.VectorSubcoreMesh(core_axis_name="c", subcore_axis_name="s")
    sc = pl.kernel(
        functools.partial(_sc_body, n),
        out_type=jax.ShapeDtypeStruct((n, HIDDEN), jnp.float32),
        mesh=mesh,
        scratch_types=[
            pltpu.VMEM((_CHUNK,), jnp.int32),
            pltpu.VMEM((NUM_EMB, HIDDEN), jnp.float32),
            pltpu.VMEM((2, 16), jnp.float32),
            pltpu.VMEM((_CHUNK, HIDDEN), jnp.float32),
        ],
    )
    out = sc(nums1, wtab, consts)
    return out.reshape(b, l, HIDDEN)


# speedup vs baseline: 45.9044x; 45.9044x over previous
import jax
import jax.numpy as jnp
import numpy as np
from jax.experimental import pallas as pl

HIDDEN = 128
NUM_EMB = 16
_POWERS = np.tile(
    np.array([[10.0 ** i for i in range(NUM_EMB)]], dtype=np.float32), (1, 8)
)
_RECIPS = np.float32(1.0) / _POWERS
_SCALES = _RECIPS * np.float32(0.0625)

_ROWS_PER_BLOCK = 160


def _tc_kernel(nums_ref, wtab_ref, pow_ref, rcp_ref, out_ref):
    rb = _ROWS_PER_BLOCK
    x = nums_ref[...].astype(jnp.float32)
    xb = jax.lax.broadcast_in_dim(x, (rb, 128, 128), (0, 1))
    xcol = xb.reshape(rb * 128, 128)
    pw = pow_ref[...]
    rc = rcp_ref[...]
    q = jnp.floor(xcol * rc)
    r = xcol - q * pw
    r = jnp.where(r == pw, jnp.float32(0.0), r)
    coeff = jnp.abs(r)
    out_ref[...] = jax.lax.dot_general(
        coeff, wtab_ref[...],
        dimension_numbers=(((1,), (0,)), ((), ())),
        preferred_element_type=jnp.float32,
    )


@jax.jit
def kernel(numbers, value_embs):
    b, l = numbers.shape
    n = b * l
    nums2d = numbers.reshape(n // 128, 128)
    wtab = jnp.tile(value_embs, (8, 1)) * (jnp.asarray(_SCALES).T * (1.0 / 8.0))
    rb = _ROWS_PER_BLOCK
    grid = (n // 128) // rb
    out = pl.pallas_call(
        _tc_kernel,
        grid=(grid,),
        in_specs=[
            pl.BlockSpec((rb, 128), lambda i: (i, 0)),
            pl.BlockSpec((128, HIDDEN), lambda i: (0, 0)),
            pl.BlockSpec((1, 128), lambda i: (0, 0)),
            pl.BlockSpec((1, 128), lambda i: (0, 0)),
        ],
        out_specs=pl.BlockSpec((rb * 128, HIDDEN), lambda i: (i, 0)),
        out_shape=jax.ShapeDtypeStruct((n, HIDDEN), jnp.float32),
    )(nums2d, wtab, jnp.asarray(_POWERS), jnp.asarray(_RECIPS))
    return out.reshape(b, l, HIDDEN)


# speedup vs baseline: 46.5527x; 1.0141x over previous
import jax
import jax.numpy as jnp
import numpy as np
from jax.experimental import pallas as pl

HIDDEN = 128
NUM_EMB = 16
_POWERS = np.tile(
    np.array([[10.0 ** i for i in range(NUM_EMB)]], dtype=np.float32), (1, 8)
)
_RECIPS = np.float32(1.0) / _POWERS
_SCALES = _RECIPS * np.float32(0.0625)

_ROWS_PER_BLOCK = 200


def _tc_kernel(nums_ref, wtab_ref, pow_ref, rcp_ref, out_ref):
    rb = _ROWS_PER_BLOCK
    x = nums_ref[...].astype(jnp.float32)
    xb = jax.lax.broadcast_in_dim(x, (rb, 128, 128), (0, 1))
    xcol = xb.reshape(rb * 128, 128)
    pw = pow_ref[...]
    rc = rcp_ref[...]
    q = jnp.floor(xcol * rc)
    r = xcol - q * pw
    r = jnp.where(r == pw, jnp.float32(0.0), r)
    coeff = jnp.abs(r)
    out_ref[...] = jax.lax.dot_general(
        coeff, wtab_ref[...],
        dimension_numbers=(((1,), (0,)), ((), ())),
        preferred_element_type=jnp.float32,
    )


@jax.jit
def kernel(numbers, value_embs):
    b, l = numbers.shape
    n = b * l
    nums2d = numbers.reshape(n // 128, 128)
    wtab = jnp.tile(value_embs, (8, 1)) * (jnp.asarray(_SCALES).T * (1.0 / 8.0))
    rb = _ROWS_PER_BLOCK
    grid = (n // 128) // rb
    out = pl.pallas_call(
        _tc_kernel,
        grid=(grid,),
        in_specs=[
            pl.BlockSpec((rb, 128), lambda i: (i, 0)),
            pl.BlockSpec((128, HIDDEN), lambda i: (0, 0)),
            pl.BlockSpec((1, 128), lambda i: (0, 0)),
            pl.BlockSpec((1, 128), lambda i: (0, 0)),
        ],
        out_specs=pl.BlockSpec((rb * 128, HIDDEN), lambda i: (i, 0)),
        out_shape=jax.ShapeDtypeStruct((n, HIDDEN), jnp.float32),
    )(nums2d, wtab, jnp.asarray(_POWERS), jnp.asarray(_RECIPS))
    return out.reshape(b, l, HIDDEN)
